# Initial kernel scaffold; baseline (speedup 1.0000x reference)
import jax
import jax.numpy as jnp
from jax.experimental import pallas as pl


def kernel(x, branch0_0_w, branch0_0_scale, branch0_0_shift, branch0_1_w, branch0_1_scale, branch0_1_shift, branch1_0_w, branch1_0_scale, branch1_0_shift, branch1_1_w, branch1_1_scale, branch1_1_shift, branch1_2_w, branch1_2_scale, branch1_2_shift, branch2_0_w, branch2_0_scale, branch2_0_shift, branch2_1_w, branch2_1_scale, branch2_1_shift, branch2_2_w, branch2_2_scale, branch2_2_shift, branch2_3_w, branch2_3_scale, branch2_3_shift, branch3_0_w, branch3_0_scale, branch3_0_shift, branch3_1_w, branch3_1_scale, branch3_1_shift, branch3_2_w, branch3_2_scale, branch3_2_shift, branch3_3_w, branch3_3_scale, branch3_3_shift, branch3_4_w, branch3_4_scale, branch3_4_shift, convlinear_0_w, convlinear_0_scale, convlinear_0_shift):
    raise NotImplementedError("write your pallas kernel here")


# speedup vs baseline: 3.0945x; 3.0945x over previous
import jax
import jax.numpy as jnp
from jax.experimental import pallas as pl
from jax.experimental.pallas import tpu as pltpu

_BRANCH_CONVS = [
    [(1, False)],
    [(1, True), (2, False)],
    [(1, True), (1, True), (3, False)],
    [(1, True), (1, True), (1, True), (4, False)],
]
_PAD = 4
_MPK, _MPPAD = 23, 11


def _window_max(v, k, axis):
    def sl(a, start, length):
        idx = [slice(None)] * a.ndim
        idx[axis] = slice(start, start + length)
        return a[tuple(idx)]

    p, s = v, 1
    while s * 2 <= k:
        n = p.shape[axis]
        p = jnp.maximum(sl(p, 0, n - s), sl(p, s, n - s))
        s *= 2
    out = v.shape[axis] - k + 1
    return jnp.maximum(sl(p, 0, out), sl(p, k - s, out))


def _make_body(H, W, C, Cout_total):
    M = H * W
    HP = H + 2 * _PAD
    MH = H + 2 * _MPPAD

    def body(*refs):
        it = iter(refs)
        x_ref = next(it)
        wf_ref, scf_ref, shf_ref = next(it), next(it), next(it)
        conv_refs = []
        for _ in range(10):
            conv_refs.append((next(it), next(it), next(it)))
        wcl_ref, sccl_ref, shcl_ref = next(it), next(it), next(it)
        out_ref = next(it)
        xpad_ref, fused_ref, cat_ref, mp_ref = next(it), next(it), next(it), next(it)

        x = x_ref[0]
        xb = x.reshape(M, C).astype(jnp.bfloat16)

        acc = jnp.dot(xb, wf_ref[...], preferred_element_type=jnp.float32)
        fused = jnp.maximum(acc * scf_ref[...] + shf_ref[...], 0.0)
        fused_ref[...] = fused.astype(jnp.bfloat16)

        ci = 0
        for bi, chain in enumerate(_BRANCH_CONVS):
            cur = fused_ref[:, bi * C:(bi + 1) * C]
            for (dil, relu) in chain:
                w_ref, sc_ref, sh_ref = conv_refs[ci]
                ci += 1
                xpad_ref[...] = jnp.zeros((HP, HP, C), jnp.bfloat16)
                xpad_ref[_PAD:_PAD + H, _PAD:_PAD + W, :] = cur.reshape(H, W, C)
                o = _PAD - dil
                acc = None
                for kh in range(3):
                    for kw in range(3):
                        tap = xpad_ref[o + kh * dil:o + kh * dil + H,
                                       o + kw * dil:o + kw * dil + W, :]
                        d = jnp.dot(tap.reshape(M, C), w_ref[kh * 3 + kw],
                                    preferred_element_type=jnp.float32)
                        acc = d if acc is None else acc + d
                y = acc * sc_ref[...] + sh_ref[...]
                if relu:
                    y = jnp.maximum(y, 0.0)
                cur = y.astype(jnp.bfloat16)
            cat_ref[:, bi * C:(bi + 1) * C] = cur

        acc = jnp.dot(cat_ref[...], wcl_ref[...], preferred_element_type=jnp.float32)
        ycl = jnp.maximum(acc * sccl_ref[...] + shcl_ref[...], 0.0)
        out_ref[0, :, :, 0:4 * C] = ycl.reshape(H, W, 4 * C)

        mp_ref[...] = jnp.full((MH, MH, C), -jnp.inf, jnp.float32)
        mp_ref[_MPPAD:_MPPAD + H, _MPPAD:_MPPAD + W, :] = x
        colmax = _window_max(mp_ref[...], _MPK, axis=0)
        mp = _window_max(colmax, _MPK, axis=1)
        out_ref[0, :, :, 4 * C:5 * C] = mp

        out_ref[0, :, :, 5 * C:6 * C] = x

    return body


def kernel(x, branch0_0_w, branch0_0_scale, branch0_0_shift, branch0_1_w, branch0_1_scale, branch0_1_shift, branch1_0_w, branch1_0_scale, branch1_0_shift, branch1_1_w, branch1_1_scale, branch1_1_shift, branch1_2_w, branch1_2_scale, branch1_2_shift, branch2_0_w, branch2_0_scale, branch2_0_shift, branch2_1_w, branch2_1_scale, branch2_1_shift, branch2_2_w, branch2_2_scale, branch2_2_shift, branch2_3_w, branch2_3_scale, branch2_3_shift, branch3_0_w, branch3_0_scale, branch3_0_shift, branch3_1_w, branch3_1_scale, branch3_1_shift, branch3_2_w, branch3_2_scale, branch3_2_shift, branch3_3_w, branch3_3_scale, branch3_3_shift, branch3_4_w, branch3_4_scale, branch3_4_shift, convlinear_0_w, convlinear_0_scale, convlinear_0_shift):
    N, C, H, W = x.shape
    xh = jnp.transpose(x, (0, 2, 3, 1))
    bf = jnp.bfloat16

    stems = [(branch0_0_w, branch0_0_scale, branch0_0_shift),
             (branch1_0_w, branch1_0_scale, branch1_0_shift),
             (branch2_0_w, branch2_0_scale, branch2_0_shift),
             (branch3_0_w, branch3_0_scale, branch3_0_shift)]
    wf = jnp.concatenate([w[0] for (w, _, _) in stems], axis=-1).astype(bf)
    scf = jnp.concatenate([s for (_, s, _) in stems], axis=-1)
    shf = jnp.concatenate([s for (_, _, s) in stems], axis=-1)

    convs = [(branch0_1_w, branch0_1_scale, branch0_1_shift),
             (branch1_1_w, branch1_1_scale, branch1_1_shift),
             (branch1_2_w, branch1_2_scale, branch1_2_shift),
             (branch2_1_w, branch2_1_scale, branch2_1_shift),
             (branch2_2_w, branch2_2_scale, branch2_2_shift),
             (branch2_3_w, branch2_3_scale, branch2_3_shift),
             (branch3_1_w, branch3_1_scale, branch3_1_shift),
             (branch3_2_w, branch3_2_scale, branch3_2_shift),
             (branch3_3_w, branch3_3_scale, branch3_3_shift),
             (branch3_4_w, branch3_4_scale, branch3_4_shift)]

    wcl = convlinear_0_w[0].astype(bf)

    operands = [xh, wf, scf, shf]
    for (w, sc, sh) in convs:
        operands += [w.astype(bf), sc, sh]
    operands += [wcl, convlinear_0_scale, convlinear_0_shift]

    def inv(a):
        nd = a.ndim
        return pl.BlockSpec(a.shape, lambda n, _nd=nd: (0,) * _nd)

    in_specs = [pl.BlockSpec((1, H, W, C), lambda n: (n, 0, 0, 0))]
    in_specs += [inv(a) for a in operands[1:]]

    HP = H + 2 * _PAD
    MH = H + 2 * _MPPAD
    out = pl.pallas_call(
        _make_body(H, W, C, 6 * C),
        out_shape=jax.ShapeDtypeStruct((N, H, W, 6 * C), jnp.float32),
        grid=(N,),
        in_specs=in_specs,
        out_specs=pl.BlockSpec((1, H, W, 6 * C), lambda n: (n, 0, 0, 0)),
        scratch_shapes=[
            pltpu.VMEM((HP, HP, C), bf),
            pltpu.VMEM((H * W, 4 * C), bf),
            pltpu.VMEM((H * W, 4 * C), bf),
            pltpu.VMEM((MH, MH, C), jnp.float32),
        ],
        compiler_params=pltpu.CompilerParams(
            dimension_semantics=("parallel",),
            vmem_limit_bytes=58 * 1024 * 1024,
        ),
    )(*operands)
    return jnp.transpose(out, (0, 3, 1, 2))
